# baseline (speedup 1.0000x reference)
import functools
import numpy as np
import jax
import jax.numpy as jnp
from jax import lax
from jax.experimental import pallas as pl
from jax.experimental.pallas import tpu as pltpu
from jax.experimental.pallas import tpu_sc as plsc

_DIFFUSION_STEPS = 1000
_BETA_START = 0.0001
_BETA_END = 0.02


def _make_tables():
    betas = np.linspace(_BETA_START, _BETA_END, _DIFFUSION_STEPS, dtype=np.float32)
    alphas = (np.float32(1.0) - betas).astype(np.float32)
    alphas_cumprod = np.cumprod(alphas, dtype=np.float32)
    sqrt_acp = np.sqrt(alphas_cumprod).astype(np.float32)
    sqrt_omacp = np.sqrt((np.float32(1.0) - alphas_cumprod)).astype(np.float32)
    return sqrt_acp, sqrt_omacp


_SQRT_ACP, _SQRT_OMACP = _make_tables()

_NC = 2
_NS = 16

_SAMPLES_PER_BLOCK = 8


def _noise_body(ts_ref, a_tab_ref, b_tab_ref, x_ref, n_ref, o_ref):
    i = pl.program_id(0)
    for s in range(_SAMPLES_PER_BLOCK):
        t = ts_ref[i * _SAMPLES_PER_BLOCK + s]
        a = a_tab_ref[t]
        b = b_tab_ref[t]
        o_ref[s] = a * x_ref[s] + b * n_ref[s]


def _sc_copy_body(n_hbm, out_hbm):
    wid = lax.axis_index("s") * _NC + lax.axis_index("c")
    pltpu.sync_copy(n_hbm.at[wid], out_hbm.at[wid])


def kernel(clean_future, timesteps, noise):
    batch, ch, h, w = clean_future.shape

    mesh = plsc.VectorSubcoreMesh(core_axis_name="c", subcore_axis_name="s")
    n_out = pl.kernel(
        _sc_copy_body,
        mesh=mesh,
        out_type=jax.ShapeDtypeStruct(clean_future.shape, jnp.float32),
        compiler_params=pltpu.CompilerParams(needs_layout_passes=False),
    )(noise)

    spb = _SAMPLES_PER_BLOCK
    block = (spb, ch, h, w)
    grid_spec = pltpu.PrefetchScalarGridSpec(
        num_scalar_prefetch=3,
        grid=(batch // spb,),
        in_specs=[
            pl.BlockSpec(block, lambda i, *_: (i, 0, 0, 0)),
            pl.BlockSpec(block, lambda i, *_: (i, 0, 0, 0)),
        ],
        out_specs=pl.BlockSpec(block, lambda i, *_: (i, 0, 0, 0)),
    )

    out = pl.pallas_call(
        _noise_body,
        grid_spec=grid_spec,
        out_shape=jax.ShapeDtypeStruct(clean_future.shape, jnp.float32),
    )(timesteps, jnp.asarray(_SQRT_ACP), jnp.asarray(_SQRT_OMACP), clean_future, noise)

    return out, n_out


# speedup vs baseline: 13.3496x; 13.3496x over previous
import functools
import numpy as np
import jax
import jax.numpy as jnp
from jax import lax
from jax.experimental import pallas as pl
from jax.experimental.pallas import tpu as pltpu
from jax.experimental.pallas import tpu_sc as plsc

_DIFFUSION_STEPS = 1000
_BETA_START = 0.0001
_BETA_END = 0.02


def _make_tables():
    betas = np.linspace(_BETA_START, _BETA_END, _DIFFUSION_STEPS, dtype=np.float32)
    alphas = (np.float32(1.0) - betas).astype(np.float32)
    alphas_cumprod = np.cumprod(alphas, dtype=np.float32)
    sqrt_acp = np.sqrt(alphas_cumprod).astype(np.float32)
    sqrt_omacp = np.sqrt((np.float32(1.0) - alphas_cumprod)).astype(np.float32)
    return sqrt_acp, sqrt_omacp


_SQRT_ACP, _SQRT_OMACP = _make_tables()

_NC = 2
_NS = 16

_SAMPLES_PER_BLOCK = 8


def _noise_body(ts_ref, a_tab_ref, b_tab_ref, x_ref, n_ref, o_ref):
    i = pl.program_id(0)
    for s in range(_SAMPLES_PER_BLOCK):
        t = ts_ref[i * _SAMPLES_PER_BLOCK + s]
        a = a_tab_ref[t]
        b = b_tab_ref[t]
        o_ref[s] = a * x_ref[s] + b * n_ref[s]


_ROWS_PER_CHUNK = 64
_NBUF = 4


def _sc_copy_body(n_hbm, out_hbm, b0, b1, b2, b3,
                  si0, si1, si2, si3, so0, so1, so2, so3):
    wid = lax.axis_index("s") * _NC + lax.axis_index("c")
    ch, h, w = n_hbm.shape[1], n_hbm.shape[2], n_hbm.shape[3]
    rpc = _ROWS_PER_CHUNK
    chunks = []
    for c in range(ch):
        for r0 in range(0, h, rpc):
            chunks.append((c, r0))
    n_chunks = len(chunks)
    bufs = (b0, b1, b2, b3)
    sis = (si0, si1, si2, si3)
    sos = (so0, so1, so2, so3)

    def start_in(ci):
        c, r0 = chunks[ci]
        s = ci % _NBUF
        return pltpu.async_copy(n_hbm.at[wid, c, pl.ds(r0, rpc), :], bufs[s], sis[s])

    def start_out(ci):
        c, r0 = chunks[ci]
        s = ci % _NBUF
        return pltpu.async_copy(bufs[s], out_hbm.at[wid, c, pl.ds(r0, rpc), :], sos[s])

    in_h = [None] * n_chunks
    out_h = [None] * n_chunks
    out_waited = [False] * n_chunks
    in_h[0] = start_in(0)
    if n_chunks > 1:
        in_h[1] = start_in(1)
    for ci in range(n_chunks):
        in_h[ci].wait()
        out_h[ci] = start_out(ci)
        nxt = ci + 2
        if nxt < n_chunks:
            prev = nxt - _NBUF
            if prev >= 0:
                out_h[prev].wait()
                out_waited[prev] = True
            in_h[nxt] = start_in(nxt)
    for ci in range(n_chunks):
        if not out_waited[ci]:
            out_h[ci].wait()


def kernel(clean_future, timesteps, noise):
    batch, ch, h, w = clean_future.shape

    mesh = plsc.VectorSubcoreMesh(core_axis_name="c", subcore_axis_name="s")
    rpc = _ROWS_PER_CHUNK
    n_out = pl.kernel(
        _sc_copy_body,
        mesh=mesh,
        out_type=jax.ShapeDtypeStruct(clean_future.shape, jnp.float32),
        scratch_types=(
            [pltpu.VMEM((rpc, w), jnp.float32) for _ in range(_NBUF)]
            + [pltpu.SemaphoreType.DMA for _ in range(2 * _NBUF)]
        ),
        compiler_params=pltpu.CompilerParams(needs_layout_passes=False),
    )(noise)

    spb = _SAMPLES_PER_BLOCK
    block = (spb, ch, h, w)
    grid_spec = pltpu.PrefetchScalarGridSpec(
        num_scalar_prefetch=3,
        grid=(batch // spb,),
        in_specs=[
            pl.BlockSpec(block, lambda i, *_: (i, 0, 0, 0)),
            pl.BlockSpec(block, lambda i, *_: (i, 0, 0, 0)),
        ],
        out_specs=pl.BlockSpec(block, lambda i, *_: (i, 0, 0, 0)),
    )

    out = pl.pallas_call(
        _noise_body,
        grid_spec=grid_spec,
        out_shape=jax.ShapeDtypeStruct(clean_future.shape, jnp.float32),
    )(timesteps, jnp.asarray(_SQRT_ACP), jnp.asarray(_SQRT_OMACP), clean_future, noise)

    return out, n_out


# speedup vs baseline: 23.2915x; 1.7447x over previous
import numpy as np
import jax
import jax.numpy as jnp
from jax.experimental import pallas as pl
from jax.experimental.pallas import tpu as pltpu

_DIFFUSION_STEPS = 1000
_BETA_START = 0.0001
_BETA_END = 0.02


def _make_tables():
    betas = np.linspace(_BETA_START, _BETA_END, _DIFFUSION_STEPS, dtype=np.float32)
    alphas = (np.float32(1.0) - betas).astype(np.float32)
    alphas_cumprod = np.cumprod(alphas, dtype=np.float32)
    sqrt_acp = np.sqrt(alphas_cumprod).astype(np.float32)
    sqrt_omacp = np.sqrt((np.float32(1.0) - alphas_cumprod)).astype(np.float32)
    return sqrt_acp, sqrt_omacp


_SQRT_ACP, _SQRT_OMACP = _make_tables()

_LANES = 128


_SAMPLES_PER_BLOCK = 8


def _noise_body(ts_ref, a_tab_ref, b_tab_ref, x_ref, n_ref, o_ref, n_out_ref):
    i = pl.program_id(0)
    for s in range(_SAMPLES_PER_BLOCK):
        t = ts_ref[i * _SAMPLES_PER_BLOCK + s]
        a = a_tab_ref[t]
        b = b_tab_ref[t]
        nv = n_ref[s]
        o_ref[s] = a * x_ref[s] + b * nv
        n_out_ref[s] = nv


def kernel(clean_future, timesteps, noise):
    batch, ch, h, w = clean_future.shape

    spb = _SAMPLES_PER_BLOCK
    hs = 2
    block = (spb, ch, h // hs, w)
    grid_spec = pltpu.PrefetchScalarGridSpec(
        num_scalar_prefetch=3,
        grid=(batch // spb, hs),
        in_specs=[
            pl.BlockSpec(block, lambda i, j, *_: (i, 0, j, 0)),
            pl.BlockSpec(block, lambda i, j, *_: (i, 0, j, 0)),
        ],
        out_specs=[
            pl.BlockSpec(block, lambda i, j, *_: (i, 0, j, 0)),
            pl.BlockSpec(block, lambda i, j, *_: (i, 0, j, 0)),
        ],
    )

    out, n_out = pl.pallas_call(
        _noise_body,
        grid_spec=grid_spec,
        out_shape=[
            jax.ShapeDtypeStruct(clean_future.shape, jnp.float32),
            jax.ShapeDtypeStruct(clean_future.shape, jnp.float32),
        ],
    )(timesteps, jnp.asarray(_SQRT_ACP), jnp.asarray(_SQRT_OMACP), clean_future, noise)

    return out, n_out
